# baseline (speedup 1.0000x reference)
import functools

import jax
import jax.numpy as jnp
from jax import lax
from jax.experimental import pallas as pl
from jax.experimental.pallas import tpu as pltpu
from jax.experimental.pallas import tpu_sc as plsc

VOCAB = 30522
D = 768
B = 128
S = 512
EPS = 1e-12
NTOK = B * S
NC = 2
NS = 16
NW = NC * NS
CH = 16
NSEG = 4
BSEG = B // NSEG
NCHUNK = BSEG
NSLOT = 4


def _sc_gather_body(ids_hbm, word_hbm, out_hbm, idx_all, w_bufs,
                    sem_w, sem_o):
    cid = lax.axis_index("c")
    sid = lax.axis_index("s")
    wid = sid * NC + cid
    pblk = wid * CH

    pltpu.sync_copy(ids_hbm.at[pl.ds(wid * BSEG * CH, BSEG * CH)], idx_all)

    def issue(c, b):
        idx = idx_all[pl.ds(c * CH, CH)]
        pltpu.async_copy(word_hbm.at[idx], w_bufs[b], sem_w[b])

    def wait_in(b):
        pltpu.make_async_copy(word_hbm.at[pl.ds(0, CH)], w_bufs[b],
                              sem_w[b]).wait()

    def wait_out(b):
        pltpu.make_async_copy(w_bufs[b], out_hbm.at[pl.ds(0, CH)],
                              sem_o[b]).wait()

    issue(0, 0)
    issue(1, 1)

    def step(c, b):
        b2 = (b + 2) % NSLOT

        @pl.when(c >= 2)
        def _():
            wait_out(b2)

        @pl.when(c + 2 < NCHUNK)
        def _():
            issue(c + 2, b2)

        wait_in(b)
        pltpu.async_copy(w_bufs[b],
                         out_hbm.at[pl.ds(c * S + pblk, CH)], sem_o[b])

    def outer(g, carry):
        for k in range(NSLOT):
            step(g * NSLOT + k, k)
        return carry

    lax.fori_loop(0, NCHUNK // NSLOT, outer, 0)
    wait_out((NCHUNK - 2) % NSLOT)
    wait_out((NCHUNK - 1) % NSLOT)


def _tc_body(wr_ref, tid_ref, pos_ref, tt_ref, gam_ref, bet_ref, out_ref):
    x = wr_ref[...]
    t0 = tt_ref[0, :][None, :]
    dt = (tt_ref[1, :] - tt_ref[0, :])[None, :]
    tid = tid_ref[...][:, None]
    x = x + pos_ref[...] + t0 + tid * dt
    mean = jnp.mean(x, axis=-1, keepdims=True)
    xc = x - mean
    var = jnp.mean(xc * xc, axis=-1, keepdims=True)
    r = lax.rsqrt(var + jnp.float32(EPS))
    out_ref[...] = xc * r * gam_ref[...][None, :] + bet_ref[...][None, :]


def kernel(input_ids, token_type_ids, word_embeddings, token_type_embeddings,
           position_embeddings, ln_gamma, ln_beta):
    mesh = plsc.VectorSubcoreMesh(core_axis_name="c", subcore_axis_name="s")
    gather = functools.partial(
        pl.kernel,
        mesh=mesh,
        out_type=jax.ShapeDtypeStruct((BSEG * S, D), jnp.float32),
        scratch_types=[
            pltpu.VMEM((BSEG * CH,), jnp.int32),
            [pltpu.VMEM((CH, D), jnp.float32) for _ in range(NSLOT)],
            [pltpu.SemaphoreType.DMA for _ in range(NSLOT)],
            [pltpu.SemaphoreType.DMA for _ in range(NSLOT)],
        ],
    )(_sc_gather_body)

    ln = pl.pallas_call(
        _tc_body,
        grid=(BSEG,),
        in_specs=[
            pl.BlockSpec((S, D), lambda i: (i, 0)),
            pl.BlockSpec((S,), lambda i: (i,)),
            pl.BlockSpec((S, D), lambda i: (0, 0)),
            pl.BlockSpec((2, D), lambda i: (0, 0)),
            pl.BlockSpec((D,), lambda i: (0,)),
            pl.BlockSpec((D,), lambda i: (0,)),
        ],
        out_specs=pl.BlockSpec((S, D), lambda i: (i, 0)),
        out_shape=jax.ShapeDtypeStruct((BSEG * S, D), jnp.float32),
    )

    ids32 = input_ids.astype(jnp.int32)
    tidf_all = token_type_ids.astype(jnp.float32)
    outs = []
    for s in range(NSEG):
        seg = ids32[s * BSEG:(s + 1) * BSEG]
        ids = (seg.reshape(BSEG, NW, CH).transpose(1, 0, 2)
               .reshape(NW * BSEG * CH))
        tidf = tidf_all[s * BSEG:(s + 1) * BSEG].reshape(BSEG * S)
        wr = gather(ids, word_embeddings)
        outs.append(ln(wr, tidf, position_embeddings, token_type_embeddings,
                       ln_gamma, ln_beta))
    return jnp.concatenate(outs, axis=0).reshape(B, S, D)
